# baseline (speedup 1.0000x reference)
import jax
import jax.numpy as jnp
from jax.experimental import pallas as pl
from jax.experimental.pallas import tpu as pltpu

_K = 25
_PAD = (_K - 1) // 2
_F = 5


def _mavg_kernel(x_ref, o_ref):
    x = x_ref[...]
    L = x.shape[0]
    top = jnp.broadcast_to(x[0:1, :], (_PAD, x.shape[1]))
    bot = jnp.broadcast_to(x[L - 1:L, :], (_PAD, x.shape[1]))
    p = jnp.concatenate([top, x, bot], axis=0)
    n5 = L + 2 * _PAD - (_F - 1)
    s5 = p[0:n5]
    for t in range(1, _F):
        s5 = s5 + p[t:t + n5]
    out = s5[0:L]
    for m in range(1, _F):
        out = out + s5[_F * m:_F * m + L]
    o_ref[...] = out * (1.0 / _K)


def kernel(x):
    B, L, C = x.shape
    return pl.pallas_call(
        _mavg_kernel,
        out_shape=jax.ShapeDtypeStruct((B, L, C), x.dtype),
        grid=(B,),
        in_specs=[pl.BlockSpec((None, L, C), lambda b: (b, 0, 0))],
        out_specs=pl.BlockSpec((None, L, C), lambda b: (b, 0, 0)),
        compiler_params=pltpu.CompilerParams(
            dimension_semantics=("parallel",),
        ),
    )(x)


# speedup vs baseline: 2.2033x; 2.2033x over previous
import jax
import jax.numpy as jnp
from jax.experimental import pallas as pl
from jax.experimental.pallas import tpu as pltpu

_K = 25
_PAD = (_K - 1) // 2


def _mavg_kernel(x_ref, o_ref):
    x = x_ref[...]
    bt, L, C = x.shape
    top = jnp.broadcast_to(x[:, 0:1, :], (bt, _PAD, C))
    bot = jnp.broadcast_to(x[:, L - 1:L, :], (bt, _PAD, C))
    p = jnp.concatenate([top, x, bot], axis=1)

    s2 = p[:, 0:L + 23, :] + p[:, 1:L + 24, :]
    s4 = s2[:, 0:L + 21, :] + s2[:, 2:L + 23, :]
    s8 = s4[:, 0:L + 17, :] + s4[:, 4:L + 21, :]
    s16 = s8[:, 0:L + 9, :] + s8[:, 8:L + 17, :]
    s24 = s16[:, 0:L, :] + s8[:, 16:L + 16, :]
    out = s24 + p[:, 24:L + 24, :]
    o_ref[...] = out * (1.0 / _K)


def kernel(x):
    B, L, C = x.shape
    bt = 8
    while B % bt:
        bt //= 2
    return pl.pallas_call(
        _mavg_kernel,
        out_shape=jax.ShapeDtypeStruct((B, L, C), x.dtype),
        grid=(B // bt,),
        in_specs=[pl.BlockSpec((bt, L, C), lambda b: (b, 0, 0))],
        out_specs=pl.BlockSpec((bt, L, C), lambda b: (b, 0, 0)),
        compiler_params=pltpu.CompilerParams(
            dimension_semantics=("parallel",),
            vmem_limit_bytes=56 * 1024 * 1024,
        ),
    )(x)


# speedup vs baseline: 2.2312x; 1.0127x over previous
import jax
import jax.numpy as jnp
from jax.experimental import pallas as pl
from jax.experimental.pallas import tpu as pltpu

_K = 25
_PAD = (_K - 1) // 2


def _mavg_kernel(x_ref, o_ref):
    x = x_ref[...]
    bt, L, C = x.shape
    top = jnp.broadcast_to(x[:, 0:1, :], (bt, _PAD, C))
    bot = jnp.broadcast_to(x[:, L - 1:L, :], (bt, _PAD, C))
    p = jnp.concatenate([top, x, bot], axis=1)

    s2 = p[:, 0:L + 23, :] + p[:, 1:L + 24, :]
    s4 = s2[:, 0:L + 21, :] + s2[:, 2:L + 23, :]
    s8 = s4[:, 0:L + 17, :] + s4[:, 4:L + 21, :]
    s16 = s8[:, 0:L + 9, :] + s8[:, 8:L + 17, :]
    s24 = s16[:, 0:L, :] + s8[:, 16:L + 16, :]
    out = s24 + p[:, 24:L + 24, :]
    o_ref[...] = out * (1.0 / _K)


def kernel(x):
    B, L, C = x.shape
    bt = 16
    while B % bt:
        bt //= 2
    return pl.pallas_call(
        _mavg_kernel,
        out_shape=jax.ShapeDtypeStruct((B, L, C), x.dtype),
        grid=(B // bt,),
        in_specs=[pl.BlockSpec((bt, L, C), lambda b: (b, 0, 0))],
        out_specs=pl.BlockSpec((bt, L, C), lambda b: (b, 0, 0)),
        compiler_params=pltpu.CompilerParams(
            dimension_semantics=("parallel",),
            vmem_limit_bytes=56 * 1024 * 1024,
        ),
    )(x)
